# baseline (speedup 1.0000x reference)
import functools

import jax
import jax.numpy as jnp
from jax import lax
from jax.experimental import pallas as pl
from jax.experimental.pallas import tpu as pltpu
from jax.experimental.pallas import tpu_sc as plsc

_NUM_CORES = 2
_NUM_SUBCORES = 16


def kernel(q, k, weight):
    q_len = q.shape[0]
    k_len = k.shape[0]
    max_len, d = weight.shape
    offset = max_len // 2 + max_len % 2

    n_workers = _NUM_CORES * _NUM_SUBCORES
    per_worker = q_len // n_workers
    slice_words = k_len * d

    mesh = plsc.VectorSubcoreMesh(core_axis_name="c", subcore_axis_name="s")

    @functools.partial(
        pl.kernel,
        out_type=jax.ShapeDtypeStruct((q_len * k_len * d,), jnp.float32),
        mesh=mesh,
        scratch_types=[
            pltpu.SemaphoreType.DMA,
        ],
    )
    def body(w_hbm, out_hbm, sem):
        c = lax.axis_index("c")
        s = lax.axis_index("s")

        wid = c * _NUM_SUBCORES + s
        copies = []
        for t in range(per_worker):
            i = wid * per_worker + t
            src_start = pl.multiple_of((offset - i) * d, d)
            dst_start = pl.multiple_of(i * slice_words, slice_words)
            copies.append(
                pltpu.async_copy(
                    w_hbm.at[pl.ds(src_start, slice_words)],
                    out_hbm.at[pl.ds(dst_start, slice_words)],
                    sem,
                )
            )
        for cp in copies:
            cp.wait()

    out_flat = body(weight.reshape(-1))
    return out_flat.reshape(q_len, k_len, d)


# speedup vs baseline: 22.4430x; 22.4430x over previous
import functools

import jax
import jax.numpy as jnp
from jax import lax
from jax.experimental import pallas as pl
from jax.experimental.pallas import tpu as pltpu
from jax.experimental.pallas import tpu_sc as plsc

_NUM_CORES = 2
_NUM_SUBCORES = 16


def kernel(q, k, weight):
    q_len = q.shape[0]
    k_len = k.shape[0]
    max_len, d = weight.shape
    offset = max_len // 2 + max_len % 2

    n_workers = _NUM_CORES * _NUM_SUBCORES
    per_worker = q_len // n_workers
    j_chunk = 128
    n_chunks = k_len // j_chunk
    win_rows = j_chunk + per_worker - 1
    chunk_words = j_chunk * d

    mesh = plsc.VectorSubcoreMesh(core_axis_name="c", subcore_axis_name="s")

    @functools.partial(
        pl.kernel,
        out_type=jax.ShapeDtypeStruct((q_len * k_len * d,), jnp.float32),
        mesh=mesh,
        scratch_types=[
            pltpu.VMEM((win_rows * d,), jnp.float32),
            pltpu.SemaphoreType.DMA,
            pltpu.SemaphoreType.DMA,
        ],
    )
    def body(w_hbm, out_hbm, buf, gsem, ssem):
        c = lax.axis_index("c")
        s = lax.axis_index("s")
        wid = c * _NUM_SUBCORES + s
        i0 = wid * per_worker

        for cix in range(n_chunks):
            j0 = cix * j_chunk
            base = pl.multiple_of((offset - i0 - (per_worker - 1) + j0) * d, d)
            pltpu.async_copy(
                w_hbm.at[pl.ds(base, win_rows * d)], buf, gsem
            ).wait()
            scatters = []
            for t in range(per_worker):
                i = i0 + t
                src_start = (per_worker - 1 - t) * d
                dst_start = pl.multiple_of(i * k_len * d + j0 * d, chunk_words)
                scatters.append(
                    pltpu.async_copy(
                        buf.at[pl.ds(src_start, chunk_words)],
                        out_hbm.at[pl.ds(dst_start, chunk_words)],
                        ssem,
                    )
                )
            for cp in scatters:
                cp.wait()

    out_flat = body(weight.reshape(-1))
    return out_flat.reshape(q_len, k_len, d)


# speedup vs baseline: 53.0175x; 2.3623x over previous
import functools

import jax
import jax.numpy as jnp
from jax import lax
from jax.experimental import pallas as pl
from jax.experimental.pallas import tpu as pltpu
from jax.experimental.pallas import tpu_sc as plsc

_NUM_CORES = 2
_NUM_SUBCORES = 16


def kernel(q, k, weight):
    q_len = q.shape[0]
    k_len = k.shape[0]
    max_len, d = weight.shape
    offset = max_len // 2 + max_len % 2

    half = q_len // _NUM_CORES
    n_cls = 8
    per_cls = half // n_cls
    per_tile = per_cls // 2
    win_rows = k_len + (per_cls - 1) * n_cls

    w_stage = jnp.stack([
        jnp.stack([
            weight[offset - c * half - (per_cls - 1) * n_cls - rho:
                   offset - c * half - (per_cls - 1) * n_cls - rho + win_rows]
            for rho in range(n_cls)
        ])
        for c in range(_NUM_CORES)
    ])

    mesh = plsc.VectorSubcoreMesh(core_axis_name="c", subcore_axis_name="s")

    @functools.partial(
        pl.kernel,
        out_type=jax.ShapeDtypeStruct((q_len, k_len, d), jnp.float32),
        mesh=mesh,
        scratch_types=[
            pltpu.VMEM_SHARED((n_cls // 2, win_rows, d), jnp.float32),
            pltpu.SemaphoreType.DMA,
        ],
    )
    def body(wst_hbm, out_hbm, rep_spmem, sem):
        c = lax.axis_index("c")
        s = lax.axis_index("s")
        rho = lax.rem(s, n_cls)
        u = lax.div(s, n_cls)
        n_slots = n_cls // 2

        for p in range(2):
            @pl.when(s < n_slots)
            def _stage():
                pltpu.sync_copy(
                    wst_hbm.at[c, p * n_slots + s], rep_spmem.at[s]
                )

            plsc.subcore_barrier()

            active = jnp.logical_and(
                rho >= p * n_slots, rho < (p + 1) * n_slots
            )

            @pl.when(active)
            def _work():
                slot = rho - p * n_slots
                copies = []
                for m in range(per_tile):
                    n = u * per_tile + m
                    i = c * half + rho + n_cls * n
                    src_start = pl.multiple_of(
                        (per_cls - 1) * n_cls - n_cls * n, 8
                    )
                    copies.append(
                        pltpu.async_copy(
                            rep_spmem.at[slot, pl.ds(src_start, k_len)],
                            out_hbm.at[i],
                            sem,
                        )
                    )
                for cp in copies:
                    cp.wait()

            plsc.subcore_barrier()

    return body(w_stage)


# speedup vs baseline: 57.5577x; 1.0856x over previous
import functools

import jax
import jax.numpy as jnp
from jax import lax
from jax.experimental import pallas as pl
from jax.experimental.pallas import tpu as pltpu
from jax.experimental.pallas import tpu_sc as plsc

_NUM_CORES = 2
_NUM_SUBCORES = 16
_SC_ROWS = 128


def kernel(q, k, weight):
    q_len = q.shape[0]
    k_len = k.shape[0]
    max_len, d = weight.shape
    offset = max_len // 2 + max_len % 2

    half = _SC_ROWS // _NUM_CORES
    n_cls = 8
    per_cls = half // n_cls
    per_tile = per_cls // 2
    win_sc = k_len + (per_cls - 1) * n_cls

    w_sc = jnp.stack([
        jnp.stack([
            weight[offset - c * half - (per_cls - 1) * n_cls - rho:
                   offset - c * half - (per_cls - 1) * n_cls - rho + win_sc]
            for rho in range(n_cls)
        ])
        for c in range(_NUM_CORES)
    ])

    mesh = plsc.VectorSubcoreMesh(core_axis_name="c", subcore_axis_name="s")

    @functools.partial(
        pl.kernel,
        out_type=jax.ShapeDtypeStruct((_SC_ROWS, k_len, d), jnp.float32),
        mesh=mesh,
        scratch_types=[
            pltpu.VMEM_SHARED((n_cls // 2, win_sc, d), jnp.float32),
            pltpu.SemaphoreType.DMA,
        ],
    )
    def sc_body(wst_hbm, out_hbm, rep_spmem, sem):
        c = lax.axis_index("c")
        s = lax.axis_index("s")
        rho = lax.rem(s, n_cls)
        u = lax.div(s, n_cls)
        n_slots = n_cls // 2

        for p in range(2):
            @pl.when(s < n_slots)
            def _stage():
                pltpu.sync_copy(
                    wst_hbm.at[c, p * n_slots + s], rep_spmem.at[s]
                )

            plsc.subcore_barrier()

            active = jnp.logical_and(
                rho >= p * n_slots, rho < (p + 1) * n_slots
            )

            @pl.when(active)
            def _work():
                slot = rho - p * n_slots
                copies = []
                for m in range(per_tile):
                    n = u * per_tile + m
                    i = c * half + rho + n_cls * n
                    src_start = pl.multiple_of(
                        (per_cls - 1) * n_cls - n_cls * n, 8
                    )
                    copies.append(
                        pltpu.async_copy(
                            rep_spmem.at[slot, pl.ds(src_start, k_len)],
                            out_hbm.at[i],
                            sem,
                        )
                    )
                for cp in copies:
                    cp.wait()

            plsc.subcore_barrier()

    sc_part = sc_body(w_sc)

    nt = (q_len - _SC_ROWS) // n_cls
    win_tc = k_len + (nt - 1) * n_cls
    w_tc = jnp.stack([
        weight[offset - _SC_ROWS - (nt - 1) * n_cls - rho:
               offset - _SC_ROWS - (nt - 1) * n_cls - rho + win_tc]
        for rho in range(n_cls)
    ])

    grid = 2
    cls_per_step = n_cls // grid

    def tc_body(w_ref, out_ref, w_vmem, sem):
        g = pl.program_id(0)
        pltpu.sync_copy(w_ref.at[pl.ds(g * cls_per_step, cls_per_step)], w_vmem)
        copies = []
        for r in range(cls_per_step):
            for n in range(nt):
                i = _SC_ROWS + g * cls_per_step + r + n_cls * n
                src_start = pl.multiple_of((nt - 1) * n_cls - n_cls * n, 8)
                copies.append(
                    pltpu.async_copy(
                        w_vmem.at[r, pl.ds(src_start, k_len)],
                        out_ref.at[i],
                        sem,
                    )
                )
        for cp in copies:
            cp.wait()

    tc_full = pl.pallas_call(
        tc_body,
        grid=(grid,),
        in_specs=[pl.BlockSpec(memory_space=pl.ANY)],
        out_specs=pl.BlockSpec(memory_space=pl.ANY),
        out_shape=jax.ShapeDtypeStruct((q_len, k_len, d), jnp.float32),
        scratch_shapes=[
            pltpu.VMEM((cls_per_step, win_tc, d), jnp.float32),
            pltpu.SemaphoreType.DMA,
        ],
        compiler_params=pltpu.CompilerParams(
            dimension_semantics=("parallel",)
        ),
    )(w_tc)

    return lax.dynamic_update_slice(tc_full, sc_part, (0, 0, 0))


# speedup vs baseline: 62.2861x; 1.0822x over previous
import functools

import jax
import jax.numpy as jnp
from jax import lax
from jax.experimental import pallas as pl
from jax.experimental.pallas import tpu as pltpu
from jax.experimental.pallas import tpu_sc as plsc

_NUM_CORES = 2
_NUM_SUBCORES = 16
_SC_ROWS = 96


def kernel(q, k, weight):
    q_len = q.shape[0]
    k_len = k.shape[0]
    max_len, d = weight.shape
    offset = max_len // 2 + max_len % 2

    half = _SC_ROWS // _NUM_CORES
    n_cls = 8
    per_cls = half // n_cls
    per_tile = per_cls // 2
    win_sc = k_len + (per_cls - 1) * n_cls

    w_sc = jnp.stack([
        jnp.stack([
            weight[offset - c * half - (per_cls - 1) * n_cls - rho:
                   offset - c * half - (per_cls - 1) * n_cls - rho + win_sc]
            for rho in range(n_cls)
        ])
        for c in range(_NUM_CORES)
    ])

    mesh = plsc.VectorSubcoreMesh(core_axis_name="c", subcore_axis_name="s")

    @functools.partial(
        pl.kernel,
        out_type=jax.ShapeDtypeStruct((_SC_ROWS, k_len, d), jnp.float32),
        mesh=mesh,
        scratch_types=[
            pltpu.VMEM_SHARED((n_cls // 2, win_sc, d), jnp.float32),
            pltpu.SemaphoreType.DMA,
        ],
    )
    def sc_body(wst_hbm, out_hbm, rep_spmem, sem):
        c = lax.axis_index("c")
        s = lax.axis_index("s")
        rho = lax.rem(s, n_cls)
        u = lax.div(s, n_cls)
        n_slots = n_cls // 2

        for p in range(2):
            @pl.when(s < n_slots)
            def _stage():
                pltpu.sync_copy(
                    wst_hbm.at[c, p * n_slots + s], rep_spmem.at[s]
                )

            plsc.subcore_barrier()

            active = jnp.logical_and(
                rho >= p * n_slots, rho < (p + 1) * n_slots
            )

            @pl.when(active)
            def _work():
                slot = rho - p * n_slots
                copies = []
                for m in range(per_tile):
                    n = u * per_tile + m
                    i = c * half + rho + n_cls * n
                    src_start = pl.multiple_of(
                        (per_cls - 1) * n_cls - n_cls * n, 8
                    )
                    copies.append(
                        pltpu.async_copy(
                            rep_spmem.at[slot, pl.ds(src_start, k_len)],
                            out_hbm.at[i],
                            sem,
                        )
                    )
                for cp in copies:
                    cp.wait()

            plsc.subcore_barrier()

    sc_part = sc_body(w_sc)

    nt = (q_len - _SC_ROWS) // n_cls
    win_tc = k_len + (nt - 1) * n_cls
    w_tc = jnp.stack([
        weight[offset - _SC_ROWS - (nt - 1) * n_cls - rho:
               offset - _SC_ROWS - (nt - 1) * n_cls - rho + win_tc]
        for rho in range(n_cls)
    ])

    grid = 2
    cls_per_step = n_cls // grid

    def tc_body(w_ref, out_ref, w_vmem, sem):
        g = pl.program_id(0)
        pltpu.sync_copy(w_ref.at[pl.ds(g * cls_per_step, cls_per_step)], w_vmem)
        copies = []
        for r in range(cls_per_step):
            for n in range(nt):
                i = _SC_ROWS + g * cls_per_step + r + n_cls * n
                src_start = pl.multiple_of((nt - 1) * n_cls - n_cls * n, 8)
                copies.append(
                    pltpu.async_copy(
                        w_vmem.at[r, pl.ds(src_start, k_len)],
                        out_ref.at[i],
                        sem,
                    )
                )
        for cp in copies:
            cp.wait()

    tc_full = pl.pallas_call(
        tc_body,
        grid=(grid,),
        in_specs=[pl.BlockSpec(memory_space=pl.ANY)],
        out_specs=pl.BlockSpec(memory_space=pl.ANY),
        out_shape=jax.ShapeDtypeStruct((q_len, k_len, d), jnp.float32),
        scratch_shapes=[
            pltpu.VMEM((cls_per_step, win_tc, d), jnp.float32),
            pltpu.SemaphoreType.DMA,
        ],
        compiler_params=pltpu.CompilerParams(
            dimension_semantics=("parallel",)
        ),
    )(w_tc)

    return lax.dynamic_update_slice(tc_full, sc_part, (0, 0, 0))


# speedup vs baseline: 67.7672x; 1.0880x over previous
import functools

import jax
import jax.numpy as jnp
from jax import lax
from jax.experimental import pallas as pl
from jax.experimental.pallas import tpu as pltpu
from jax.experimental.pallas import tpu_sc as plsc

_NUM_CORES = 2
_NUM_SUBCORES = 16
_SC_ROWS = 64


def kernel(q, k, weight):
    q_len = q.shape[0]
    k_len = k.shape[0]
    max_len, d = weight.shape
    offset = max_len // 2 + max_len % 2

    half = _SC_ROWS // _NUM_CORES
    n_cls = 8
    per_cls = half // n_cls
    per_tile = per_cls // 2
    win_sc = k_len + (per_cls - 1) * n_cls

    w_sc = jnp.stack([
        jnp.stack([
            weight[offset - c * half - (per_cls - 1) * n_cls - rho:
                   offset - c * half - (per_cls - 1) * n_cls - rho + win_sc]
            for rho in range(n_cls)
        ])
        for c in range(_NUM_CORES)
    ])

    mesh = plsc.VectorSubcoreMesh(core_axis_name="c", subcore_axis_name="s")

    @functools.partial(
        pl.kernel,
        out_type=jax.ShapeDtypeStruct((_SC_ROWS, k_len, d), jnp.float32),
        mesh=mesh,
        scratch_types=[
            pltpu.VMEM_SHARED((n_cls // 2, win_sc, d), jnp.float32),
            pltpu.SemaphoreType.DMA,
        ],
    )
    def sc_body(wst_hbm, out_hbm, rep_spmem, sem):
        c = lax.axis_index("c")
        s = lax.axis_index("s")
        rho = lax.rem(s, n_cls)
        u = lax.div(s, n_cls)
        n_slots = n_cls // 2

        for p in range(2):
            @pl.when(s < n_slots)
            def _stage():
                pltpu.sync_copy(
                    wst_hbm.at[c, p * n_slots + s], rep_spmem.at[s]
                )

            plsc.subcore_barrier()

            active = jnp.logical_and(
                rho >= p * n_slots, rho < (p + 1) * n_slots
            )

            @pl.when(active)
            def _work():
                slot = rho - p * n_slots
                copies = []
                for m in range(per_tile):
                    n = u * per_tile + m
                    i = c * half + rho + n_cls * n
                    src_start = pl.multiple_of(
                        (per_cls - 1) * n_cls - n_cls * n, 8
                    )
                    copies.append(
                        pltpu.async_copy(
                            rep_spmem.at[slot, pl.ds(src_start, k_len)],
                            out_hbm.at[i],
                            sem,
                        )
                    )
                for cp in copies:
                    cp.wait()

            plsc.subcore_barrier()

    sc_part = sc_body(w_sc)

    nt = (q_len - _SC_ROWS) // n_cls
    win_tc = k_len + (nt - 1) * n_cls
    w_tc = jnp.stack([
        weight[offset - _SC_ROWS - (nt - 1) * n_cls - rho:
               offset - _SC_ROWS - (nt - 1) * n_cls - rho + win_tc]
        for rho in range(n_cls)
    ])

    grid = 2
    cls_per_step = n_cls // grid

    def tc_body(w_ref, out_ref, w_vmem, sem):
        g = pl.program_id(0)
        pltpu.sync_copy(w_ref.at[pl.ds(g * cls_per_step, cls_per_step)], w_vmem)
        copies = []
        for r in range(cls_per_step):
            for n in range(nt):
                i = _SC_ROWS + g * cls_per_step + r + n_cls * n
                src_start = pl.multiple_of((nt - 1) * n_cls - n_cls * n, 8)
                copies.append(
                    pltpu.async_copy(
                        w_vmem.at[r, pl.ds(src_start, k_len)],
                        out_ref.at[i],
                        sem,
                    )
                )
        for cp in copies:
            cp.wait()

    tc_full = pl.pallas_call(
        tc_body,
        grid=(grid,),
        in_specs=[pl.BlockSpec(memory_space=pl.ANY)],
        out_specs=pl.BlockSpec(memory_space=pl.ANY),
        out_shape=jax.ShapeDtypeStruct((q_len, k_len, d), jnp.float32),
        scratch_shapes=[
            pltpu.VMEM((cls_per_step, win_tc, d), jnp.float32),
            pltpu.SemaphoreType.DMA,
        ],
        compiler_params=pltpu.CompilerParams(
            dimension_semantics=("parallel",)
        ),
    )(w_tc)

    return lax.dynamic_update_slice(tc_full, sc_part, (0, 0, 0))


# speedup vs baseline: 73.5537x; 1.0854x over previous
import functools

import jax
import jax.numpy as jnp
from jax import lax
from jax.experimental import pallas as pl
from jax.experimental.pallas import tpu as pltpu
from jax.experimental.pallas import tpu_sc as plsc

_NUM_CORES = 2
_NUM_SUBCORES = 16
_SC_ROWS = 32


def kernel(q, k, weight):
    q_len = q.shape[0]
    k_len = k.shape[0]
    max_len, d = weight.shape
    offset = max_len // 2 + max_len % 2

    half = _SC_ROWS // _NUM_CORES
    n_cls = 8
    per_cls = half // n_cls
    per_tile = per_cls // 2
    win_sc = k_len + (per_cls - 1) * n_cls

    w_sc = jnp.stack([
        jnp.stack([
            weight[offset - c * half - (per_cls - 1) * n_cls - rho:
                   offset - c * half - (per_cls - 1) * n_cls - rho + win_sc]
            for rho in range(n_cls)
        ])
        for c in range(_NUM_CORES)
    ])

    mesh = plsc.VectorSubcoreMesh(core_axis_name="c", subcore_axis_name="s")

    @functools.partial(
        pl.kernel,
        out_type=jax.ShapeDtypeStruct((_SC_ROWS, k_len, d), jnp.float32),
        mesh=mesh,
        scratch_types=[
            pltpu.VMEM_SHARED((n_cls // 2, win_sc, d), jnp.float32),
            pltpu.SemaphoreType.DMA,
        ],
    )
    def sc_body(wst_hbm, out_hbm, rep_spmem, sem):
        c = lax.axis_index("c")
        s = lax.axis_index("s")
        rho = lax.rem(s, n_cls)
        u = lax.div(s, n_cls)
        n_slots = n_cls // 2

        for p in range(2):
            @pl.when(s < n_slots)
            def _stage():
                pltpu.sync_copy(
                    wst_hbm.at[c, p * n_slots + s], rep_spmem.at[s]
                )

            plsc.subcore_barrier()

            active = jnp.logical_and(
                rho >= p * n_slots, rho < (p + 1) * n_slots
            )

            @pl.when(active)
            def _work():
                slot = rho - p * n_slots
                copies = []
                for m in range(per_tile):
                    n = u * per_tile + m
                    i = c * half + rho + n_cls * n
                    src_start = pl.multiple_of(
                        (per_cls - 1) * n_cls - n_cls * n, 8
                    )
                    copies.append(
                        pltpu.async_copy(
                            rep_spmem.at[slot, pl.ds(src_start, k_len)],
                            out_hbm.at[i],
                            sem,
                        )
                    )
                for cp in copies:
                    cp.wait()

            plsc.subcore_barrier()

    sc_part = sc_body(w_sc)

    nt = (q_len - _SC_ROWS) // n_cls
    win_tc = k_len + (nt - 1) * n_cls
    w_tc = jnp.stack([
        weight[offset - _SC_ROWS - (nt - 1) * n_cls - rho:
               offset - _SC_ROWS - (nt - 1) * n_cls - rho + win_tc]
        for rho in range(n_cls)
    ])

    grid = 2
    cls_per_step = n_cls // grid

    def tc_body(w_ref, out_ref, w_vmem, sem):
        g = pl.program_id(0)
        pltpu.sync_copy(w_ref.at[pl.ds(g * cls_per_step, cls_per_step)], w_vmem)
        copies = []
        for r in range(cls_per_step):
            for n in range(nt):
                i = _SC_ROWS + g * cls_per_step + r + n_cls * n
                src_start = pl.multiple_of((nt - 1) * n_cls - n_cls * n, 8)
                copies.append(
                    pltpu.async_copy(
                        w_vmem.at[r, pl.ds(src_start, k_len)],
                        out_ref.at[i],
                        sem,
                    )
                )
        for cp in copies:
            cp.wait()

    tc_full = pl.pallas_call(
        tc_body,
        grid=(grid,),
        in_specs=[pl.BlockSpec(memory_space=pl.ANY)],
        out_specs=pl.BlockSpec(memory_space=pl.ANY),
        out_shape=jax.ShapeDtypeStruct((q_len, k_len, d), jnp.float32),
        scratch_shapes=[
            pltpu.VMEM((cls_per_step, win_tc, d), jnp.float32),
            pltpu.SemaphoreType.DMA,
        ],
        compiler_params=pltpu.CompilerParams(
            dimension_semantics=("parallel",)
        ),
    )(w_tc)

    return lax.dynamic_update_slice(tc_full, sc_part, (0, 0, 0))


# speedup vs baseline: 76.7185x; 1.0430x over previous
import functools

import jax
import jax.numpy as jnp
from jax import lax
from jax.experimental import pallas as pl
from jax.experimental.pallas import tpu as pltpu
from jax.experimental.pallas import tpu_sc as plsc

_NUM_CORES = 2
_NUM_SUBCORES = 16
_SC_ROWS = 16


def kernel(q, k, weight):
    q_len = q.shape[0]
    k_len = k.shape[0]
    max_len, d = weight.shape
    offset = max_len // 2 + max_len % 2

    half = _SC_ROWS // _NUM_CORES
    n_cls = 8
    per_cls = half // n_cls
    per_tile = max(per_cls // 2, 1)
    win_sc = k_len + (per_cls - 1) * n_cls

    w_sc = jnp.stack([
        jnp.stack([
            weight[offset - c * half - (per_cls - 1) * n_cls - rho:
                   offset - c * half - (per_cls - 1) * n_cls - rho + win_sc]
            for rho in range(n_cls)
        ])
        for c in range(_NUM_CORES)
    ])

    mesh = plsc.VectorSubcoreMesh(core_axis_name="c", subcore_axis_name="s")

    @functools.partial(
        pl.kernel,
        out_type=jax.ShapeDtypeStruct((_SC_ROWS, k_len, d), jnp.float32),
        mesh=mesh,
        scratch_types=[
            pltpu.VMEM_SHARED((n_cls // 2, win_sc, d), jnp.float32),
            pltpu.SemaphoreType.DMA,
        ],
    )
    def sc_body(wst_hbm, out_hbm, rep_spmem, sem):
        c = lax.axis_index("c")
        s = lax.axis_index("s")
        rho = lax.rem(s, n_cls)
        u = lax.div(s, n_cls)
        n_slots = n_cls // 2

        for p in range(2):
            @pl.when(s < n_slots)
            def _stage():
                pltpu.sync_copy(
                    wst_hbm.at[c, p * n_slots + s], rep_spmem.at[s]
                )

            plsc.subcore_barrier()

            active = jnp.logical_and(
                rho >= p * n_slots, rho < (p + 1) * n_slots
            )
            active = jnp.logical_and(active, u * per_tile < per_cls)

            @pl.when(active)
            def _work():
                slot = rho - p * n_slots
                copies = []
                for m in range(per_tile):
                    n = u * per_tile + m
                    i = c * half + rho + n_cls * n
                    src_start = pl.multiple_of(
                        (per_cls - 1) * n_cls - n_cls * n, 8
                    )
                    copies.append(
                        pltpu.async_copy(
                            rep_spmem.at[slot, pl.ds(src_start, k_len)],
                            out_hbm.at[i],
                            sem,
                        )
                    )
                for cp in copies:
                    cp.wait()

            plsc.subcore_barrier()

    sc_part = sc_body(w_sc)

    nt = (q_len - _SC_ROWS) // n_cls
    win_tc = k_len + (nt - 1) * n_cls
    w_tc = jnp.stack([
        weight[offset - _SC_ROWS - (nt - 1) * n_cls - rho:
               offset - _SC_ROWS - (nt - 1) * n_cls - rho + win_tc]
        for rho in range(n_cls)
    ])

    grid = 2
    cls_per_step = n_cls // grid

    def tc_body(w_ref, out_ref, w_vmem, sem):
        g = pl.program_id(0)
        pltpu.sync_copy(w_ref.at[pl.ds(g * cls_per_step, cls_per_step)], w_vmem)
        copies = []
        for r in range(cls_per_step):
            for n in range(nt):
                i = _SC_ROWS + g * cls_per_step + r + n_cls * n
                src_start = pl.multiple_of((nt - 1) * n_cls - n_cls * n, 8)
                copies.append(
                    pltpu.async_copy(
                        w_vmem.at[r, pl.ds(src_start, k_len)],
                        out_ref.at[i],
                        sem,
                    )
                )
        for cp in copies:
            cp.wait()

    tc_full = pl.pallas_call(
        tc_body,
        grid=(grid,),
        in_specs=[pl.BlockSpec(memory_space=pl.ANY)],
        out_specs=pl.BlockSpec(memory_space=pl.ANY),
        out_shape=jax.ShapeDtypeStruct((q_len, k_len, d), jnp.float32),
        scratch_shapes=[
            pltpu.VMEM((cls_per_step, win_tc, d), jnp.float32),
            pltpu.SemaphoreType.DMA,
        ],
        compiler_params=pltpu.CompilerParams(
            dimension_semantics=("parallel",)
        ),
    )(w_tc)

    return lax.dynamic_update_slice(tc_full, sc_part, (0, 0, 0))
